# baseline (speedup 1.0000x reference)
import dataclasses

import jax
import jax.numpy as jnp
from jax.experimental import pallas as pl
from jax.experimental.pallas import tpu as pltpu
from jax.experimental.pallas import tpu_sc as plsc

_NUM_CLASSES = 1000
_SMOOTHING = 0.1
_BATCH = 16384

_NUM_TECS = 32
_ROWS_PER_TEC = _BATCH // _NUM_TECS
_CHUNK_ROWS = 32
_CHUNK = _CHUNK_ROWS * _NUM_CLASSES
_NUM_CHUNKS = _ROWS_PER_TEC // _CHUNK_ROWS

_SV = _SMOOTHING / (_NUM_CLASSES - 1)
_HIT = (1.0 - _SMOOTHING) + _SV


def _fill_bufs(buf0, buf1):
    svv = jnp.full((16,), _SV, jnp.float32)

    @pl.loop(0, _CHUNK, step=64)
    def _(c):
        for buf in (buf0, buf1):
            buf[pl.ds(c, 16)] = svv
            buf[pl.ds(c + 16, 16)] = svv
            buf[pl.ds(c + 32, 16)] = svv
            buf[pl.ds(c + 48, 16)] = svv


def _scatter_chunk(buf, idx_buf, chunk, value):
    val = jnp.full((16,), value, jnp.float32)
    lanes = jax.lax.iota(jnp.int32, 16)
    for g in range(_CHUNK_ROWS // 16):
        rows = lanes + g * 16
        idxv = idx_buf[pl.ds(chunk * _CHUNK_ROWS + g * 16, 16)]
        plsc.store_scatter(buf, [rows * _NUM_CLASSES + idxv], val)


def _body(x_hbm, o_hbm, buf0, buf1, idx_buf, sem0, sem1):
    c = jax.lax.axis_index("c")
    s = jax.lax.axis_index("s")
    tec = c * 16 + s
    row0 = tec * _ROWS_PER_TEC

    pltpu.sync_copy(x_hbm.at[pl.ds(row0, _ROWS_PER_TEC)], idx_buf)
    _fill_bufs(buf0, buf1)

    def out_copy(buf, chunk, sem):
        start = (row0 + chunk * _CHUNK_ROWS) * _NUM_CLASSES
        return pltpu.make_async_copy(buf, o_hbm.at[pl.ds(start, _CHUNK)], sem)

    @pl.loop(0, _NUM_CHUNKS // 2)
    def _(j):
        for buf, sem, chunk in ((buf0, sem0, 2 * j), (buf1, sem1, 2 * j + 1)):
            @pl.when(j > 0)
            def _():
                prev = chunk - 2
                out_copy(buf, prev, sem).wait()
                _scatter_chunk(buf, idx_buf, prev, _SV)

            _scatter_chunk(buf, idx_buf, chunk, _HIT)
            out_copy(buf, chunk, sem).start()

    out_copy(buf0, _NUM_CHUNKS - 2, sem0).wait()
    out_copy(buf1, _NUM_CHUNKS - 1, sem1).wait()


_cp = pltpu.CompilerParams()
if "needs_layout_passes" in pltpu.CompilerParams.__dataclass_fields__:
    _cp = dataclasses.replace(_cp, needs_layout_passes=False)


@jax.jit
def kernel(x_i):
    run = pl.kernel(
        _body,
        compiler_params=_cp,
        out_type=jax.ShapeDtypeStruct((_BATCH * _NUM_CLASSES,), jnp.float32),
        mesh=plsc.VectorSubcoreMesh(core_axis_name="c", subcore_axis_name="s"),
        scratch_types=[
            pltpu.VMEM((_CHUNK,), jnp.float32),
            pltpu.VMEM((_CHUNK,), jnp.float32),
            pltpu.VMEM((_ROWS_PER_TEC,), jnp.int32),
            pltpu.SemaphoreType.DMA,
            pltpu.SemaphoreType.DMA,
        ],
    )
    return run(x_i.astype(jnp.int32)).reshape(_BATCH, _NUM_CLASSES)


# speedup vs baseline: 6.9540x; 6.9540x over previous
import jax
import jax.numpy as jnp
from jax.experimental import pallas as pl

_NUM_CLASSES = 1000
_SMOOTHING = 0.1
_BATCH = 16384
_CLS_BLOCK = 200


def _body(idx_ref, out_ref):
    sv = jnp.float32(_SMOOTHING / (_NUM_CLASSES - 1))
    hit = jnp.float32(1.0 - _SMOOTHING) + sv
    b = pl.program_id(0)
    classes = (
        jax.lax.broadcasted_iota(jnp.int32, (_CLS_BLOCK, _BATCH), 0)
        + b * _CLS_BLOCK
    )
    out_ref[...] = jnp.where(classes == idx_ref[...], hit, sv)


@jax.jit
def kernel(x_i):
    idx2d = x_i.astype(jnp.int32).reshape(1, _BATCH)
    out_t = pl.pallas_call(
        _body,
        grid=(_NUM_CLASSES // _CLS_BLOCK,),
        in_specs=[pl.BlockSpec((1, _BATCH), lambda i: (0, 0))],
        out_specs=pl.BlockSpec((_CLS_BLOCK, _BATCH), lambda i: (i, 0)),
        out_shape=jax.ShapeDtypeStruct((_NUM_CLASSES, _BATCH), jnp.float32),
    )(idx2d)
    return out_t.T
